# Initial kernel scaffold; baseline (speedup 1.0000x reference)
import functools

import jax
import jax.numpy as jnp
from jax import lax
from jax.experimental import pallas as pl
from jax.experimental.pallas import tpu as pltpu
from jax.experimental.pallas import tpu_sc as plsc

N = 4096
NNZ = 167772

NC = 2
NS = 16
LANES = 16

W = 10496
NNZ_PAD = NS * W
ENT = W // LANES

CHUNK_ROWS = 256
CHUNK = CHUNK_ROWS * N
PASSES = (N // NC) // CHUNK_ROWS
TS = CHUNK // NS
ZW = 32768


def _body(rows_hbm, cols_hbm, vals_hbm, out_hbm,
          rows_v, cols_v, vals_v, idx_b, val_b, zero_b, acc):
    c = lax.axis_index("c")
    s = lax.axis_index("s")
    shard = s * W

    pltpu.sync_copy(rows_hbm.at[pl.ds(shard, W)], rows_v)
    pltpu.sync_copy(cols_hbm.at[pl.ds(shard, W)], cols_v)
    pltpu.sync_copy(vals_hbm.at[pl.ds(shard, W)], vals_v)

    zvec = jnp.zeros((LANES,), jnp.float32)

    def zb_body(i, carry):
        zero_b[pl.ds(i * LANES, LANES)] = zvec
        return carry

    lax.fori_loop(0, ZW // LANES, zb_body, 0)

    iota = lax.iota(jnp.int32, LANES)

    for p in range(PASSES):
        base = c * (N // NC) + p * CHUNK_ROWS

        for z in range(TS // ZW):
            pltpu.sync_copy(zero_b, acc.at[pl.ds(s * TS + z * ZW, ZW)])
        plsc.subcore_barrier()

        def scan_body(i, carry):
            off = i * LANES
            row = rows_v[pl.ds(off, LANES)]
            col = cols_v[pl.ds(off, LANES)]
            val = vals_v[pl.ds(off, LANES)]
            rel = row - base
            m = (rel >= 0) & (rel < CHUNK_ROWS)
            lidx = (rel << 12) + col
            pad = off + iota
            idx_b[pl.ds(off, LANES)] = jnp.where(m, lidx, pad)
            val_b[pl.ds(off, LANES)] = jnp.where(m, val, 0.0)
            return carry

        lax.fori_loop(0, ENT, scan_body, 0)

        pltpu.sync_copy(val_b, acc.at[idx_b], add=True)
        plsc.subcore_barrier()

        pltpu.sync_copy(acc.at[pl.ds(s * TS, TS)],
                        out_hbm.at[pl.ds(base * N + s * TS, TS)])
        plsc.subcore_barrier()


_mesh = plsc.VectorSubcoreMesh(core_axis_name="c", subcore_axis_name="s",
                               num_cores=NC, num_subcores=NS)

_sc_call = functools.partial(
    pl.kernel,
    out_type=jax.ShapeDtypeStruct((N * N,), jnp.float32),
    mesh=_mesh,
    scratch_types=[
        pltpu.VMEM((W,), jnp.int32),
        pltpu.VMEM((W,), jnp.int32),
        pltpu.VMEM((W,), jnp.float32),
        pltpu.VMEM((W,), jnp.int32),
        pltpu.VMEM((W,), jnp.float32),
        pltpu.VMEM((ZW,), jnp.float32),
        pltpu.VMEM_SHARED((CHUNK,), jnp.float32),
    ],
)(_body)


def kernel(indices, values):
    rows = indices[0].astype(jnp.int32)
    cols = indices[1].astype(jnp.int32)
    vals = values.astype(jnp.float32)
    pad = NNZ_PAD - rows.shape[0]
    if pad:
        rows = jnp.concatenate([rows, jnp.zeros((pad,), jnp.int32)])
        cols = jnp.concatenate([cols, jnp.zeros((pad,), jnp.int32)])
        vals = jnp.concatenate([vals, jnp.zeros((pad,), jnp.float32)])
    out_flat = _sc_call(rows, cols, vals)
    return out_flat.reshape(N, N)


# speedup vs baseline: 4.6313x; 4.6313x over previous
import functools

import jax
import jax.numpy as jnp
from jax import lax
from jax.experimental import pallas as pl
from jax.experimental.pallas import tpu as pltpu
from jax.experimental.pallas import tpu_sc as plsc

N = 4096
NNZ = 167772

NC = 2
NS = 16
LANES = 16

W = 10496
NNZ_PAD = NS * W
ENT = W // LANES

CHUNK_ROWS = 128
CHUNK = CHUNK_ROWS * N
PASSES = (N // NC) // CHUNK_ROWS
TS = CHUNK // NS
ZW = 32768


def _body(rows_hbm, cols_hbm, vals_hbm, out_hbm,
          rows_v, cols_v, vals_v, idx_b, val_b, zero_b, acc):
    c = lax.axis_index("c")
    s = lax.axis_index("s")
    shard = s * W

    pltpu.sync_copy(rows_hbm.at[pl.ds(shard, W)], rows_v)
    pltpu.sync_copy(cols_hbm.at[pl.ds(shard, W)], cols_v)
    pltpu.sync_copy(vals_hbm.at[pl.ds(shard, W)], vals_v)

    zvec = jnp.zeros((LANES,), jnp.float32)

    def zb_body(i, carry):
        zero_b[pl.ds(i * LANES, LANES)] = zvec
        return carry

    lax.fori_loop(0, ZW // LANES, zb_body, 0)

    iota = lax.iota(jnp.int32, LANES)

    for p in range(PASSES):
        base = c * (N // NC) + p * CHUNK_ROWS

        for z in range(TS // ZW):
            pltpu.sync_copy(zero_b, acc.at[pl.ds(s * TS + z * ZW, ZW)])
        plsc.subcore_barrier()

        def scan_body(i, carry):
            off = i * LANES
            row = rows_v[pl.ds(off, LANES)]
            col = cols_v[pl.ds(off, LANES)]
            val = vals_v[pl.ds(off, LANES)]
            rel = row - base
            m = (rel >= 0) & (rel < CHUNK_ROWS)
            lidx = (rel << 12) + col
            pad = off + iota
            idx_b[pl.ds(off, LANES)] = jnp.where(m, lidx, pad)
            val_b[pl.ds(off, LANES)] = jnp.where(m, val, 0.0)
            return carry

        lax.fori_loop(0, ENT, scan_body, 0)

        pltpu.sync_copy(val_b, acc.at[idx_b], add=True)
        plsc.subcore_barrier()

        pltpu.sync_copy(acc.at[pl.ds(s * TS, TS)],
                        out_hbm.at[pl.ds(base * N + s * TS, TS)])
        plsc.subcore_barrier()


_mesh = plsc.VectorSubcoreMesh(core_axis_name="c", subcore_axis_name="s",
                               num_cores=NC, num_subcores=NS)

_sc_call = functools.partial(
    pl.kernel,
    out_type=jax.ShapeDtypeStruct((N * N,), jnp.float32),
    mesh=_mesh,
    scratch_types=[
        pltpu.VMEM((W,), jnp.int32),
        pltpu.VMEM((W,), jnp.int32),
        pltpu.VMEM((W,), jnp.float32),
        pltpu.VMEM((W,), jnp.int32),
        pltpu.VMEM((W,), jnp.float32),
        pltpu.VMEM((ZW,), jnp.float32),
        pltpu.VMEM_SHARED((CHUNK,), jnp.float32),
    ],
)(_body)


def kernel(indices, values):
    rows = indices[0].astype(jnp.int32)
    cols = indices[1].astype(jnp.int32)
    vals = values.astype(jnp.float32)
    pad = NNZ_PAD - rows.shape[0]
    if pad:
        rows = jnp.concatenate([rows, jnp.zeros((pad,), jnp.int32)])
        cols = jnp.concatenate([cols, jnp.zeros((pad,), jnp.int32)])
        vals = jnp.concatenate([vals, jnp.zeros((pad,), jnp.float32)])
    out_flat = _sc_call(rows, cols, vals)
    return out_flat.reshape(N, N)


# speedup vs baseline: 5.1829x; 1.1191x over previous
import functools

import jax
import jax.numpy as jnp
from jax import lax
from jax.experimental import pallas as pl
from jax.experimental.pallas import tpu as pltpu
from jax.experimental.pallas import tpu_sc as plsc

N = 4096
NNZ = 167772

NC = 2
NS = 16
LANES = 16

W = 10496
NNZ_PAD = NS * W
ENT = W // LANES

PASS_ROWS = [176] * 11 + [112]
CHUNK = max(PASS_ROWS) * N
ZW = 32768


def _body(rows_hbm, cols_hbm, vals_hbm, out_hbm,
          rows_v, cols_v, vals_v, idx_b, val_b, zero_b, acc):
    c = lax.axis_index("c")
    s = lax.axis_index("s")
    shard = s * W

    pltpu.sync_copy(rows_hbm.at[pl.ds(shard, W)], rows_v)
    pltpu.sync_copy(cols_hbm.at[pl.ds(shard, W)], cols_v)
    pltpu.sync_copy(vals_hbm.at[pl.ds(shard, W)], vals_v)

    zvec = jnp.zeros((LANES,), jnp.float32)

    def zb_body(i, carry):
        zero_b[pl.ds(i * LANES, LANES)] = zvec
        return carry

    lax.fori_loop(0, ZW // LANES, zb_body, 0)

    iota = lax.iota(jnp.int32, LANES)

    row_off = 0
    for rows_p in PASS_ROWS:
        base = c * (N // NC) + row_off
        row_off += rows_p
        ts = rows_p * N // NS

        zdone = 0
        while zdone < ts:
            zn = min(ZW, ts - zdone)
            pltpu.sync_copy(zero_b.at[pl.ds(0, zn)],
                            acc.at[pl.ds(s * ts + zdone, zn)])
            zdone += zn
        plsc.subcore_barrier()

        def scan_body(i, carry):
            off = i * LANES
            row = rows_v[pl.ds(off, LANES)]
            col = cols_v[pl.ds(off, LANES)]
            val = vals_v[pl.ds(off, LANES)]
            rel = row - base
            m = (rel >= 0) & (rel < rows_p)
            lidx = (rel << 12) + col
            pad = off + iota
            idx_b[pl.ds(off, LANES)] = jnp.where(m, lidx, pad)
            val_b[pl.ds(off, LANES)] = jnp.where(m, val, 0.0)
            return carry

        lax.fori_loop(0, ENT, scan_body, 0)

        pltpu.sync_copy(val_b, acc.at[idx_b], add=True)
        plsc.subcore_barrier()

        pltpu.sync_copy(acc.at[pl.ds(s * ts, ts)],
                        out_hbm.at[pl.ds(base * N + s * ts, ts)])
        plsc.subcore_barrier()


_mesh = plsc.VectorSubcoreMesh(core_axis_name="c", subcore_axis_name="s",
                               num_cores=NC, num_subcores=NS)

_sc_call = functools.partial(
    pl.kernel,
    out_type=jax.ShapeDtypeStruct((N * N,), jnp.float32),
    mesh=_mesh,
    scratch_types=[
        pltpu.VMEM((W,), jnp.int32),
        pltpu.VMEM((W,), jnp.int32),
        pltpu.VMEM((W,), jnp.float32),
        pltpu.VMEM((W,), jnp.int32),
        pltpu.VMEM((W,), jnp.float32),
        pltpu.VMEM((ZW,), jnp.float32),
        pltpu.VMEM_SHARED((CHUNK,), jnp.float32),
    ],
)(_body)


def kernel(indices, values):
    rows = indices[0].astype(jnp.int32)
    cols = indices[1].astype(jnp.int32)
    vals = values.astype(jnp.float32)
    pad = NNZ_PAD - rows.shape[0]
    if pad:
        rows = jnp.concatenate([rows, jnp.zeros((pad,), jnp.int32)])
        cols = jnp.concatenate([cols, jnp.zeros((pad,), jnp.int32)])
        vals = jnp.concatenate([vals, jnp.zeros((pad,), jnp.float32)])
    out_flat = _sc_call(rows, cols, vals)
    return out_flat.reshape(N, N)


# speedup vs baseline: 5.6720x; 1.0944x over previous
import functools

import jax
import jax.numpy as jnp
from jax import lax
from jax.experimental import pallas as pl
from jax.experimental.pallas import tpu as pltpu
from jax.experimental.pallas import tpu_sc as plsc

N = 4096
NNZ = 167772

NC = 2
NS = 16
LANES = 16

W = 10496
NNZ_PAD = NS * W
ENT = W // LANES

PASS_ROWS = [176] * 11 + [112]
CHUNK = max(PASS_ROWS) * N
ZW = 32768


def _body(rows_hbm, cols_hbm, vals_hbm, out_hbm,
          rows_v, cols_v, vals_v, idx_b, zero_b, acc):
    c = lax.axis_index("c")
    s = lax.axis_index("s")
    shard = s * W

    pltpu.sync_copy(rows_hbm.at[pl.ds(shard, W)], rows_v)
    pltpu.sync_copy(cols_hbm.at[pl.ds(shard, W)], cols_v)
    pltpu.sync_copy(vals_hbm.at[pl.ds(shard, W)], vals_v)

    zvec = jnp.zeros((LANES,), jnp.float32)

    def zb_body(i, carry):
        zero_b[pl.ds(i * LANES, LANES)] = zvec
        return carry

    lax.fori_loop(0, ZW // LANES, zb_body, 0)

    row_off = 0
    for rows_p in PASS_ROWS:
        base = c * (N // NC) + row_off
        row_off += rows_p
        ts = rows_p * N // NS

        zdone = 0
        while zdone < ts:
            zn = min(ZW, ts - zdone)
            pltpu.sync_copy(zero_b.at[pl.ds(0, zn)],
                            acc.at[pl.ds(s * ts + zdone, zn)])
            zdone += zn
        plsc.subcore_barrier()

        def scan_body(i, carry):
            off = i * LANES
            row = rows_v[pl.ds(off, LANES)]
            col = cols_v[pl.ds(off, LANES)]
            rel = row - base
            m = (rel >= 0) & (rel < rows_p)
            idx_b[pl.ds(off, LANES)] = jnp.where(m, (rel << 12) + col, -1)
            return carry

        lax.fori_loop(0, ENT, scan_body, 0)

        pltpu.sync_copy(vals_v,
                        acc.at[plsc.Indices(idx_b, ignored_value=-1)],
                        add=True)
        plsc.subcore_barrier()

        pltpu.sync_copy(acc.at[pl.ds(s * ts, ts)],
                        out_hbm.at[pl.ds(base * N + s * ts, ts)])
        plsc.subcore_barrier()


_mesh = plsc.VectorSubcoreMesh(core_axis_name="c", subcore_axis_name="s",
                               num_cores=NC, num_subcores=NS)

_sc_call = functools.partial(
    pl.kernel,
    out_type=jax.ShapeDtypeStruct((N * N,), jnp.float32),
    mesh=_mesh,
    scratch_types=[
        pltpu.VMEM((W,), jnp.int32),
        pltpu.VMEM((W,), jnp.int32),
        pltpu.VMEM((W,), jnp.float32),
        pltpu.VMEM((W,), jnp.int32),
        pltpu.VMEM((ZW,), jnp.float32),
        pltpu.VMEM_SHARED((CHUNK,), jnp.float32),
    ],
)(_body)


def kernel(indices, values):
    rows = indices[0].astype(jnp.int32)
    cols = indices[1].astype(jnp.int32)
    vals = values.astype(jnp.float32)
    pad = NNZ_PAD - rows.shape[0]
    if pad:
        rows = jnp.concatenate([rows, jnp.full((pad,), 0, jnp.int32)])
        cols = jnp.concatenate([cols, jnp.full((pad,), 0, jnp.int32)])
        vals = jnp.concatenate([vals, jnp.zeros((pad,), jnp.float32)])
    out_flat = _sc_call(rows, cols, vals)
    return out_flat.reshape(N, N)


# speedup vs baseline: 7.8710x; 1.3877x over previous
import functools

import jax
import jax.numpy as jnp
from jax import lax
from jax.experimental import pallas as pl
from jax.experimental.pallas import tpu as pltpu
from jax.experimental.pallas import tpu_sc as plsc

N = 4096
NNZ = 167772

NC = 2
NS = 16
LANES = 16

W = 10496
NNZ_PAD = NS * W
ENT = W // LANES

PASS_ROWS = [176] * 11 + [112]
CHUNK = max(PASS_ROWS) * N
ZW = 32768


def _body(rows_hbm, cols_hbm, vals_hbm, out_hbm,
          rows_v, cols_v, vals_v, idx_b, zero_b, acc, sem):
    c = lax.axis_index("c")
    s = lax.axis_index("s")
    shard = s * W

    pltpu.sync_copy(rows_hbm.at[pl.ds(shard, W)], rows_v)
    pltpu.sync_copy(cols_hbm.at[pl.ds(shard, W)], cols_v)
    pltpu.sync_copy(vals_hbm.at[pl.ds(shard, W)], vals_v)

    zvec = jnp.zeros((LANES,), jnp.float32)

    def zb_body(i, carry):
        zero_b[pl.ds(i * LANES, LANES)] = zvec
        return carry

    lax.fori_loop(0, ZW // LANES, zb_body, 0)

    acc_f = acc
    row_off = 0
    for rows_p in PASS_ROWS:
        base = c * (N // NC) + row_off
        row_off += rows_p
        ts = rows_p * N // NS

        zdone = 0
        while zdone < ts:
            zn = min(ZW, ts - zdone)
            pltpu.sync_copy(zero_b.at[pl.ds(0, zn)],
                            acc_f.at[pl.ds(s * ts + zdone, zn)])
            zdone += zn
        plsc.subcore_barrier()

        def scan_body(i, carry):
            off = i * LANES
            row = rows_v[pl.ds(off, LANES)]
            col = cols_v[pl.ds(off, LANES)]
            rel = row - base
            m = (rel >= 0) & (rel < rows_p)
            idx_b[pl.ds(off, LANES)] = jnp.where(m, (rel << 12) + col, -1)
            return carry

        lax.fori_loop(0, ENT, scan_body, 0)

        pltpu.sync_copy(vals_v,
                        acc_f.at[plsc.Indices(idx_b, ignored_value=-1)],
                        add=True)
        plsc.subcore_barrier()

        nr = rows_p // NS
        cps = [pltpu.async_copy(acc_f.at[pl.ds((s * nr + r) * N, N)],
                                out_hbm.at[base + s * nr + r, :], sem)
               for r in range(nr)]
        for cp in cps:
            cp.wait()
        plsc.subcore_barrier()


_mesh = plsc.VectorSubcoreMesh(core_axis_name="c", subcore_axis_name="s",
                               num_cores=NC, num_subcores=NS)

_sc_call = functools.partial(
    pl.kernel,
    out_type=jax.ShapeDtypeStruct((N, N), jnp.float32),
    mesh=_mesh,
    scratch_types=[
        pltpu.VMEM((W,), jnp.int32),
        pltpu.VMEM((W,), jnp.int32),
        pltpu.VMEM((W,), jnp.float32),
        pltpu.VMEM((W,), jnp.int32),
        pltpu.VMEM((ZW,), jnp.float32),
        pltpu.VMEM_SHARED((CHUNK,), jnp.float32),
        pltpu.SemaphoreType.DMA,
    ],
)(_body)


def kernel(indices, values):
    rows = indices[0].astype(jnp.int32)
    cols = indices[1].astype(jnp.int32)
    vals = values.astype(jnp.float32)
    pad = NNZ_PAD - rows.shape[0]
    if pad:
        rows = jnp.concatenate([rows, jnp.full((pad,), 0, jnp.int32)])
        cols = jnp.concatenate([cols, jnp.full((pad,), 0, jnp.int32)])
        vals = jnp.concatenate([vals, jnp.zeros((pad,), jnp.float32)])
    return _sc_call(rows, cols, vals)
